# baseline (speedup 1.0000x reference)
import functools

import jax
import jax.numpy as jnp
from jax import lax
from jax.experimental import pallas as pl
from jax.experimental.pallas import tpu as pltpu
from jax.experimental.pallas import tpu_sc as plsc

_B, _S, _D = 2, 2048, 1024
_E, _K, _F = 8, 2, 4096
_N = _B * _S
_P = _N * _K

_BM = 512
_NB = _P // _BM + _E
_PAD = _NB * _BM
_FB = 1024
_NF = _F // _FB

_BT = 512

_NC, _NS = 2, 16
_NW = _NC * _NS
_GCH = 64
_CT = 32


def _router_body(x_ref, wg_ref, idx_ref, w_ref):
    logits = jnp.dot(x_ref[...], wg_ref[...],
                     preferred_element_type=jnp.float32)
    col = lax.broadcasted_iota(jnp.int32, logits.shape, 1)
    m1 = jnp.max(logits, axis=1, keepdims=True)
    i1 = jnp.min(jnp.where(logits == m1, col, _E), axis=1, keepdims=True)
    l2 = jnp.where(col == i1, -jnp.inf, logits)
    m2 = jnp.max(l2, axis=1, keepdims=True)
    i2 = jnp.min(jnp.where(l2 == m2, col, _E), axis=1, keepdims=True)
    t = jnp.exp(m2 - m1)
    w1 = 1.0 / (1.0 + t)
    w2 = t / (1.0 + t)
    idx_ref[...] = jnp.where(col == 0, i1, jnp.where(col == 1, i2, 0))
    w_ref[...] = jnp.where(col == 0, w1, jnp.where(col == 1, w2, 0.0))


def _router(xt, wg):
    return pl.pallas_call(
        _router_body,
        grid=(_N // _BT,),
        in_specs=[pl.BlockSpec((_BT, _D), lambda i: (i, 0)),
                  pl.BlockSpec((_D, _E), lambda i: (0, 0))],
        out_specs=[pl.BlockSpec((_BT, _E), lambda i: (i, 0)),
                   pl.BlockSpec((_BT, _E), lambda i: (i, 0))],
        out_shape=[jax.ShapeDtypeStruct((_N, _E), jnp.int32),
                   jax.ShapeDtypeStruct((_N, _E), jnp.float32)],
    )(xt, wg)


def _mlp_body(be_ref, x_ref, w1_ref, w2_ref, b1_ref, b2_ref, ws_ref, y_ref):
    i, j = pl.program_id(0), pl.program_id(1)
    e = be_ref[i]
    xb = x_ref[...]
    h = jnp.dot(xb, w1_ref[0], preferred_element_type=jnp.float32)
    h += b1_ref[e, pl.ds(j * _FB, _FB)][None, :]
    h = jax.nn.gelu(h)
    y = jnp.dot(h.astype(jnp.bfloat16), w2_ref[0],
                preferred_element_type=jnp.float32)

    @pl.when(j == 0)
    def _():
        y_ref[...] = jnp.broadcast_to(b2_ref[e, :][None, :], y_ref.shape)

    y_ref[...] += y

    @pl.when(j == _NF - 1)
    def _():
        ws = ws_ref[pl.ds(i * _BM, _BM)]
        y_ref[...] *= ws[:, None]


def _mlp(block_expert, xs, w1, w2, b1, b2, ws):
    grid_spec = pltpu.PrefetchScalarGridSpec(
        num_scalar_prefetch=1,
        grid=(_NB, _NF),
        in_specs=[
            pl.BlockSpec((_BM, _D), lambda i, j, be: (i, 0)),
            pl.BlockSpec((1, _D, _FB), lambda i, j, be: (be[i], 0, j)),
            pl.BlockSpec((1, _FB, _D), lambda i, j, be: (be[i], j, 0)),
            pl.BlockSpec((_E, _F), lambda i, j, be: (0, 0)),
            pl.BlockSpec((_E, _D), lambda i, j, be: (0, 0)),
            pl.BlockSpec((_PAD,), lambda i, j, be: (0,)),
        ],
        out_specs=pl.BlockSpec((_BM, _D), lambda i, j, be: (i, 0)),
    )
    return pl.pallas_call(
        _mlp_body,
        grid_spec=grid_spec,
        out_shape=jax.ShapeDtypeStruct((_PAD, _D), jnp.float32),
        compiler_params=pltpu.CompilerParams(
            dimension_semantics=("arbitrary", "arbitrary")),
    )(block_expert, xs, w1, w2, b1, b2, ws)


def _sc_gather(x32, src):
    rows_per_w = _PAD // _NW
    ch = 96
    nch = rows_per_w // ch
    w = _D // 2
    mesh = plsc.VectorSubcoreMesh(core_axis_name="c", subcore_axis_name="s")

    @functools.partial(
        pl.kernel, mesh=mesh,
        out_type=jax.ShapeDtypeStruct((_PAD, w), jnp.int32),
        scratch_types=[pltpu.VMEM((rows_per_w,), jnp.int32),
                       pltpu.VMEM((ch, w), jnp.int32),
                       pltpu.VMEM((ch, w), jnp.int32),
                       pltpu.SemaphoreType.DMA,
                       pltpu.SemaphoreType.DMA,
                       pltpu.SemaphoreType.DMA,
                       pltpu.SemaphoreType.DMA])
    def gk(x_hbm, src_hbm, out_hbm, idx_v, buf0, buf1, g0, g1, s0, s1):
        wid = lax.axis_index("s") * _NC + lax.axis_index("c")
        base = wid * rows_per_w
        pltpu.sync_copy(src_hbm.at[pl.ds(base, rows_per_w)], idx_v)
        bufs, gsems, wsems = (buf0, buf1), (g0, g1), (s0, s1)

        def start_gather(c):
            return pltpu.async_copy(
                x_hbm.at[idx_v.at[pl.ds(c * ch, ch)]], bufs[c % 2],
                gsems[c % 2])

        def start_write(c):
            return pltpu.async_copy(
                bufs[c % 2], out_hbm.at[pl.ds(base + c * ch, ch)],
                wsems[c % 2])

        gh, wh = {}, {}
        gh[0] = start_gather(0)
        gh[1] = start_gather(1)
        gh[0].wait()
        wh[0] = start_write(0)
        gh[1].wait()
        wh[1] = start_write(1)
        for c in range(2, nch):
            wh[c - 2].wait()
            gh[c] = start_gather(c)
            gh[c].wait()
            wh[c] = start_write(c)
        wh[nch - 2].wait()
        wh[nch - 1].wait()

    return gk(x32, src)


def _sc_combine(ys, slots):
    tok_per_w = _N // _NW
    nch = tok_per_w // _CT
    mesh = plsc.VectorSubcoreMesh(core_axis_name="c", subcore_axis_name="s")

    @functools.partial(
        pl.kernel, mesh=mesh,
        out_type=jax.ShapeDtypeStruct((_N, _D), jnp.float32),
        scratch_types=[pltpu.VMEM((2 * _CT,), jnp.int32),
                       pltpu.VMEM((2 * _CT, _D), jnp.float32),
                       pltpu.VMEM((_CT, _D), jnp.float32),
                       pltpu.SemaphoreType.DMA])
    def ck(ys_hbm, sl_hbm, out_hbm, idx_v, rows_v, out_v, sem):
        wid = lax.axis_index("s") * _NC + lax.axis_index("c")
        t0 = wid * tok_per_w

        def chunk(c, carry):
            tb = t0 + c * _CT
            pltpu.sync_copy(sl_hbm.at[pl.ds(2 * tb, 2 * _CT)], idx_v)
            pltpu.async_copy(ys_hbm.at[idx_v], rows_v, sem).wait()

            def tok(r, carry2):
                def lane(q, carry3):
                    s = pl.ds(q * 16, 16)
                    out_v[r, s] = rows_v[2 * r, s] + rows_v[2 * r + 1, s]
                    return carry3
                return lax.fori_loop(0, _D // 16, lane, carry2)

            lax.fori_loop(0, _CT, tok, 0)
            pltpu.sync_copy(out_v, out_hbm.at[pl.ds(tb, _CT)])
            return carry

        lax.fori_loop(0, nch, chunk, 0)

    return ck(ys, slots)


def _dispatch_metadata(idx8, w8):
    eflat = idx8[:, :_K].reshape(-1)
    wflat = w8[:, :_K].reshape(-1)
    order = jnp.argsort(eflat, stable=True)
    e_sorted = eflat[order]
    counts = jnp.sum(eflat[:, None] == jnp.arange(_E)[None, :], axis=0)
    padded = ((counts + _BM - 1) // _BM) * _BM
    ends = jnp.cumsum(padded)
    starts = ends - padded
    tight_ends = jnp.cumsum(counts)
    tight_starts = tight_ends - counts
    ranks = jnp.arange(_P) - tight_starts[e_sorted]
    pos_sorted = (starts[e_sorted] + ranks).astype(jnp.int32)
    src = jnp.zeros((_PAD,), jnp.int32).at[pos_sorted].set(
        (order // _K).astype(jnp.int32))
    ws = jnp.zeros((_PAD,), jnp.float32).at[pos_sorted].set(wflat[order])
    slot = jnp.zeros((_P,), jnp.int32).at[order].set(pos_sorted)
    block_expert = jnp.minimum(
        jnp.searchsorted(ends, jnp.arange(_NB) * _BM, side="right"),
        _E - 1).astype(jnp.int32)
    return src, ws, block_expert, slot


def kernel(x, Wg, W1, b1, W2, b2):
    orig_shape = x.shape
    xt = x.reshape(_N, _D)
    idx8, w8 = _router(xt, Wg)
    src, ws, block_expert, slot = _dispatch_metadata(idx8, w8)
    x32 = lax.bitcast_convert_type(
        xt.astype(jnp.bfloat16).reshape(_N, _D // 2, 2), jnp.int32)
    xs32 = _sc_gather(x32, src)
    xs = lax.bitcast_convert_type(xs32, jnp.bfloat16).reshape(_PAD, _D)
    ys = _mlp(block_expert, xs, W1.astype(jnp.bfloat16),
              W2.astype(jnp.bfloat16), b1, b2, ws)
    out = _sc_combine(ys, slot)
    return out.reshape(orig_shape)


# speedup vs baseline: 1.6081x; 1.6081x over previous
import functools

import jax
import jax.numpy as jnp
from jax import lax
from jax.experimental import pallas as pl
from jax.experimental.pallas import tpu as pltpu
from jax.experimental.pallas import tpu_sc as plsc

_B, _S, _D = 2, 2048, 1024
_E, _K, _F = 8, 2, 4096
_N = _B * _S
_P = _N * _K

_BM = 256
_NB = _P // _BM + _E
_PAD = _NB * _BM

_BT = 512

_NC, _NS = 2, 16
_NW = _NC * _NS
_GCH = 64
_CT = 32


def _router_body(x_ref, wg_ref, idx_ref, w_ref):
    logits = jnp.dot(x_ref[...], wg_ref[...],
                     preferred_element_type=jnp.float32)
    col = lax.broadcasted_iota(jnp.int32, logits.shape, 1)
    m1 = jnp.max(logits, axis=1, keepdims=True)
    i1 = jnp.min(jnp.where(logits == m1, col, _E), axis=1, keepdims=True)
    l2 = jnp.where(col == i1, -jnp.inf, logits)
    m2 = jnp.max(l2, axis=1, keepdims=True)
    i2 = jnp.min(jnp.where(l2 == m2, col, _E), axis=1, keepdims=True)
    t = jnp.exp(m2 - m1)
    w1 = 1.0 / (1.0 + t)
    w2 = t / (1.0 + t)
    idx_ref[...] = jnp.where(col == 0, i1, jnp.where(col == 1, i2, 0))
    w_ref[...] = jnp.where(col == 0, w1, jnp.where(col == 1, w2, 0.0))


def _router(xt, wg):
    return pl.pallas_call(
        _router_body,
        grid=(_N // _BT,),
        in_specs=[pl.BlockSpec((_BT, _D), lambda i: (i, 0)),
                  pl.BlockSpec((_D, _E), lambda i: (0, 0))],
        out_specs=[pl.BlockSpec((_BT, _E), lambda i: (i, 0)),
                   pl.BlockSpec((_BT, _E), lambda i: (i, 0))],
        out_shape=[jax.ShapeDtypeStruct((_N, _E), jnp.int32),
                   jax.ShapeDtypeStruct((_N, _E), jnp.float32)],
    )(xt, wg)


def _mlp_body(be_ref, act_ref, x_ref, w1_ref, w2_ref, b1_ref, b2_ref,
              ws_ref, y_ref):
    i = pl.program_id(0)

    @pl.when(act_ref[i] != 0)
    def _():
        e = be_ref[i]
        x = x_ref[...].astype(jnp.bfloat16)
        acc = jnp.broadcast_to(b2_ref[e, :][None, :], (_BM, _D))
        for jj in range(_F // 1024):
            sl = slice(jj * 1024, (jj + 1) * 1024)
            h = jnp.dot(x, w1_ref[0, :, sl],
                        preferred_element_type=jnp.float32)
            h += b1_ref[e, sl][None, :]
            h = jax.nn.gelu(h)
            acc = acc + jnp.dot(h.astype(jnp.bfloat16), w2_ref[0, sl, :],
                                preferred_element_type=jnp.float32)
        ws = ws_ref[pl.ds(i * _BM, _BM)]
        y_ref[...] = acc * ws[:, None]


def _mlp(block_expert, active, xs, w1, w2, b1, b2, ws):
    grid_spec = pltpu.PrefetchScalarGridSpec(
        num_scalar_prefetch=2,
        grid=(_NB,),
        in_specs=[
            pl.BlockSpec((_BM, _D), lambda i, be, act: (i, 0)),
            pl.BlockSpec((1, _D, _F), lambda i, be, act: (be[i], 0, 0)),
            pl.BlockSpec((1, _F, _D), lambda i, be, act: (be[i], 0, 0)),
            pl.BlockSpec((_E, _F), lambda i, be, act: (0, 0)),
            pl.BlockSpec((_E, _D), lambda i, be, act: (0, 0)),
            pl.BlockSpec((_PAD,), lambda i, be, act: (0,)),
        ],
        out_specs=pl.BlockSpec((_BM, _D), lambda i, be, act: (i, 0)),
    )
    return pl.pallas_call(
        _mlp_body,
        grid_spec=grid_spec,
        out_shape=jax.ShapeDtypeStruct((_PAD, _D), jnp.float32),
        compiler_params=pltpu.CompilerParams(
            dimension_semantics=("arbitrary",)),
    )(block_expert, active, xs, w1, w2, b1, b2, ws)


def _sc_gather(xb, src):
    rows_per_w = _PAD // _NW
    ch = 40
    nch = rows_per_w // ch
    nbuf = 3
    mesh = plsc.VectorSubcoreMesh(core_axis_name="c", subcore_axis_name="s")

    @functools.partial(
        pl.kernel, mesh=mesh,
        out_type=jax.ShapeDtypeStruct((_PAD, _D), jnp.float32),
        scratch_types=[pltpu.VMEM((rows_per_w,), jnp.int32)]
        + [pltpu.VMEM((ch, _D), jnp.float32)] * nbuf
        + [pltpu.SemaphoreType.DMA] * (2 * nbuf))
    def gk(x_hbm, src_hbm, out_hbm, idx_v, b0, b1, b2,
           g0, g1, g2, s0, s1, s2):
        wid = lax.axis_index("s") * _NC + lax.axis_index("c")
        base = wid * rows_per_w
        pltpu.sync_copy(src_hbm.at[pl.ds(base, rows_per_w)], idx_v)
        bufs, gsems, wsems = (b0, b1, b2), (g0, g1, g2), (s0, s1, s2)

        def start_gather(c):
            return pltpu.async_copy(
                x_hbm.at[idx_v.at[pl.ds(c * ch, ch)]], bufs[c % nbuf],
                gsems[c % nbuf])

        def start_write(c):
            return pltpu.async_copy(
                bufs[c % nbuf], out_hbm.at[pl.ds(base + c * ch, ch)],
                wsems[c % nbuf])

        gh, wh = {}, {}
        for c in range(min(nbuf, nch)):
            gh[c] = start_gather(c)
        for c in range(nch):
            gh[c].wait()
            wh[c] = start_write(c)
            if c + nbuf < nch:
                wh[c].wait()
                gh[c + nbuf] = start_gather(c + nbuf)
        for c in range(max(0, nch - nbuf), nch):
            wh[c].wait()

    return gk(xb, src)


def _sc_combine(ys, slots):
    tok_per_w = _N // _NW
    nch = tok_per_w // _CT
    mesh = plsc.VectorSubcoreMesh(core_axis_name="c", subcore_axis_name="s")

    @functools.partial(
        pl.kernel, mesh=mesh,
        out_type=jax.ShapeDtypeStruct((_N, _D), jnp.float32),
        scratch_types=[pltpu.VMEM((2 * _CT,), jnp.int32),
                       pltpu.VMEM((2 * _CT, _D), jnp.float32),
                       pltpu.VMEM((_CT, _D), jnp.float32),
                       pltpu.SemaphoreType.DMA])
    def ck(ys_hbm, sl_hbm, out_hbm, idx_v, rows_v, out_v, sem):
        wid = lax.axis_index("s") * _NC + lax.axis_index("c")
        t0 = wid * tok_per_w

        def chunk(c, carry):
            tb = t0 + c * _CT
            pltpu.sync_copy(sl_hbm.at[pl.ds(2 * tb, 2 * _CT)], idx_v)
            pltpu.async_copy(ys_hbm.at[idx_v], rows_v, sem).wait()

            def tok(r, carry2):
                def lane(q, carry3):
                    s = pl.ds(q * 16, 16)
                    out_v[r, s] = rows_v[2 * r, s] + rows_v[2 * r + 1, s]
                    return carry3
                return lax.fori_loop(0, _D // 16, lane, carry2)

            lax.fori_loop(0, _CT, tok, 0)
            pltpu.sync_copy(out_v, out_hbm.at[pl.ds(tb, _CT)])
            return carry

        lax.fori_loop(0, nch, chunk, 0)

    return ck(ys, slots)


def _dispatch_metadata(idx8, w8):
    eflat = idx8[:, :_K].reshape(-1)
    wflat = w8[:, :_K].reshape(-1)
    order = jnp.argsort(eflat, stable=True)
    e_sorted = eflat[order]
    counts = jnp.sum(eflat[:, None] == jnp.arange(_E)[None, :], axis=0)
    padded = ((counts + _BM - 1) // _BM) * _BM
    ends = jnp.cumsum(padded)
    starts = ends - padded
    tight_ends = jnp.cumsum(counts)
    tight_starts = tight_ends - counts
    ranks = jnp.arange(_P) - tight_starts[e_sorted]
    pos_sorted = (starts[e_sorted] + ranks).astype(jnp.int32)
    src = jnp.zeros((_PAD,), jnp.int32).at[pos_sorted].set(
        (order // _K).astype(jnp.int32))
    ws = jnp.zeros((_PAD,), jnp.float32).at[pos_sorted].set(wflat[order])
    slot = jnp.zeros((_P,), jnp.int32).at[order].set(pos_sorted)
    block_expert = jnp.minimum(
        jnp.searchsorted(ends, jnp.arange(_NB) * _BM, side="right"),
        _E - 1).astype(jnp.int32)
    active = (jnp.arange(_NB) * _BM
              < (starts + counts)[block_expert]).astype(jnp.int32)
    return src, ws, block_expert, active, slot


def kernel(x, Wg, W1, b1, W2, b2):
    orig_shape = x.shape
    xt = x.reshape(_N, _D)
    idx8, w8 = _router(xt, Wg)
    src, ws, block_expert, active, slot = _dispatch_metadata(idx8, w8)
    xs = _sc_gather(xt, src)
    ys = _mlp(block_expert, active, xs, W1.astype(jnp.bfloat16),
              W2.astype(jnp.bfloat16), b1, b2, ws)
    out = _sc_combine(ys, slot)
    return out.reshape(orig_shape)


# speedup vs baseline: 1.6981x; 1.0560x over previous
import functools

import jax
import jax.numpy as jnp
from jax import lax
from jax.experimental import pallas as pl
from jax.experimental.pallas import tpu as pltpu
from jax.experimental.pallas import tpu_sc as plsc

_B, _S, _D = 2, 2048, 1024
_E, _K, _F = 8, 2, 4096
_N = _B * _S
_P = _N * _K

_BM = 256
_NB = _P // _BM + _E
_PAD = _NB * _BM

_BT = 512

_NC, _NS = 2, 16
_NW = _NC * _NS
_GCH = 64
_CT = 32


def _router_body(x_ref, wg_ref, idx_ref, w_ref):
    logits = jnp.dot(x_ref[...], wg_ref[...],
                     preferred_element_type=jnp.float32)
    col = lax.broadcasted_iota(jnp.int32, logits.shape, 1)
    m1 = jnp.max(logits, axis=1, keepdims=True)
    i1 = jnp.min(jnp.where(logits == m1, col, _E), axis=1, keepdims=True)
    l2 = jnp.where(col == i1, -jnp.inf, logits)
    m2 = jnp.max(l2, axis=1, keepdims=True)
    i2 = jnp.min(jnp.where(l2 == m2, col, _E), axis=1, keepdims=True)
    t = jnp.exp(m2 - m1)
    w1 = 1.0 / (1.0 + t)
    w2 = t / (1.0 + t)
    idx_ref[...] = jnp.where(col == 0, i1, jnp.where(col == 1, i2, 0))
    w_ref[...] = jnp.where(col == 0, w1, jnp.where(col == 1, w2, 0.0))


def _router(xt, wg):
    return pl.pallas_call(
        _router_body,
        grid=(_N // _BT,),
        in_specs=[pl.BlockSpec((_BT, _D), lambda i: (i, 0)),
                  pl.BlockSpec((_D, _E), lambda i: (0, 0))],
        out_specs=[pl.BlockSpec((_BT, _E), lambda i: (i, 0)),
                   pl.BlockSpec((_BT, _E), lambda i: (i, 0))],
        out_shape=[jax.ShapeDtypeStruct((_N, _E), jnp.int32),
                   jax.ShapeDtypeStruct((_N, _E), jnp.float32)],
    )(xt, wg)


def _mlp_body(be_ref, act_ref, x_ref, w1_ref, w2_ref, b1_ref, b2_ref,
              ws_ref, y_ref):
    i = pl.program_id(0)

    @pl.when(act_ref[i] != 0)
    def _():
        e = be_ref[i]
        x = x_ref[...].astype(jnp.bfloat16)
        acc = jnp.broadcast_to(b2_ref[e, :][None, :], (_BM, _D))
        for jj in range(_F // 1024):
            sl = slice(jj * 1024, (jj + 1) * 1024)
            h = jnp.dot(x, w1_ref[0, :, sl],
                        preferred_element_type=jnp.float32)
            h += b1_ref[e, sl][None, :]
            h = jax.nn.gelu(h)
            acc = acc + jnp.dot(h.astype(jnp.bfloat16), w2_ref[0, sl, :],
                                preferred_element_type=jnp.float32)
        ws = ws_ref[pl.ds(i * _BM, _BM)]
        y_ref[...] = acc * ws[:, None]


def _mlp(block_expert, active, xs, w1, w2, b1, b2, ws):
    grid_spec = pltpu.PrefetchScalarGridSpec(
        num_scalar_prefetch=2,
        grid=(_NB,),
        in_specs=[
            pl.BlockSpec((_BM, _D), lambda i, be, act: (i, 0)),
            pl.BlockSpec((1, _D, _F), lambda i, be, act: (be[i], 0, 0)),
            pl.BlockSpec((1, _F, _D), lambda i, be, act: (be[i], 0, 0)),
            pl.BlockSpec((_E, _F), lambda i, be, act: (0, 0)),
            pl.BlockSpec((_E, _D), lambda i, be, act: (0, 0)),
            pl.BlockSpec((_PAD,), lambda i, be, act: (0,)),
        ],
        out_specs=pl.BlockSpec((_BM, _D), lambda i, be, act: (i, 0)),
    )
    return pl.pallas_call(
        _mlp_body,
        grid_spec=grid_spec,
        out_shape=jax.ShapeDtypeStruct((_PAD, _D), jnp.float32),
        compiler_params=pltpu.CompilerParams(
            dimension_semantics=("arbitrary",)),
    )(block_expert, active, xs, w1, w2, b1, b2, ws)


def _sc_gather(xb, src):
    rows_per_w = _PAD // _NW
    ch = 40
    nch = rows_per_w // ch
    nbuf = 3
    mesh = plsc.VectorSubcoreMesh(core_axis_name="c", subcore_axis_name="s")

    @functools.partial(
        pl.kernel, mesh=mesh,
        out_type=jax.ShapeDtypeStruct((_PAD, _D), jnp.float32),
        scratch_types=[pltpu.VMEM((rows_per_w,), jnp.int32)]
        + [pltpu.VMEM((ch, _D), jnp.float32)] * nbuf
        + [pltpu.SemaphoreType.DMA] * (2 * nbuf))
    def gk(x_hbm, src_hbm, out_hbm, idx_v, b0, b1, b2,
           g0, g1, g2, s0, s1, s2):
        wid = lax.axis_index("s") * _NC + lax.axis_index("c")
        base = wid * rows_per_w
        pltpu.sync_copy(src_hbm.at[pl.ds(base, rows_per_w)], idx_v)
        bufs, gsems, wsems = (b0, b1, b2), (g0, g1, g2), (s0, s1, s2)

        def start_gather(c):
            return pltpu.async_copy(
                x_hbm.at[idx_v.at[pl.ds(c * ch, ch)]], bufs[c % nbuf],
                gsems[c % nbuf])

        def start_write(c):
            return pltpu.async_copy(
                bufs[c % nbuf], out_hbm.at[pl.ds(base + c * ch, ch)],
                wsems[c % nbuf])

        gh, wh = {}, {}
        for c in range(min(nbuf, nch)):
            gh[c] = start_gather(c)
        for c in range(nch):
            gh[c].wait()
            wh[c] = start_write(c)
            if c + nbuf < nch:
                wh[c].wait()
                gh[c + nbuf] = start_gather(c + nbuf)
        for c in range(max(0, nch - nbuf), nch):
            wh[c].wait()

    return gk(xb, src)


def _sc_combine(ys, slots):
    tok_per_w = _N // _NW
    nch = tok_per_w // _CT
    mesh = plsc.VectorSubcoreMesh(core_axis_name="c", subcore_axis_name="s")

    @functools.partial(
        pl.kernel, mesh=mesh,
        out_type=jax.ShapeDtypeStruct((_N, _D), jnp.float32),
        scratch_types=[pltpu.VMEM((2 * _CT,), jnp.int32),
                       pltpu.VMEM((2 * _CT, _D), jnp.float32),
                       pltpu.VMEM((_CT, _D), jnp.float32),
                       pltpu.SemaphoreType.DMA])
    def ck(ys_hbm, sl_hbm, out_hbm, idx_v, rows_v, out_v, sem):
        wid = lax.axis_index("s") * _NC + lax.axis_index("c")
        t0 = wid * tok_per_w

        def chunk(c, carry):
            tb = t0 + c * _CT
            pltpu.sync_copy(sl_hbm.at[pl.ds(2 * tb, 2 * _CT)], idx_v)
            pltpu.async_copy(ys_hbm.at[idx_v], rows_v, sem).wait()

            def tok(r, carry2):
                def lane(q, carry3):
                    s = pl.ds(q * 16, 16)
                    out_v[r, s] = rows_v[2 * r, s] + rows_v[2 * r + 1, s]
                    return carry3
                return lax.fori_loop(0, _D // 16, lane, carry2)

            lax.fori_loop(0, _CT, tok, 0)
            pltpu.sync_copy(out_v, out_hbm.at[pl.ds(tb, _CT)])
            return carry

        lax.fori_loop(0, nch, chunk, 0)

    return ck(ys, slots)


def _dispatch_metadata(idx8, w8):
    eflat = idx8[:, :_K].reshape(-1)
    wflat = w8[:, :_K].reshape(-1)
    onehot = (eflat[:, None] == jnp.arange(_E)[None, :]).astype(jnp.int32)
    cum = jnp.cumsum(onehot, axis=0)
    counts = cum[-1]
    padded = ((counts + _BM - 1) // _BM) * _BM
    ends = jnp.cumsum(padded)
    starts = ends - padded
    rank = jnp.take_along_axis(cum, eflat[:, None], axis=1)[:, 0] - 1
    pos = (starts[eflat] + rank).astype(jnp.int32)
    src = jnp.zeros((_PAD,), jnp.int32).at[pos].set(
        (jnp.arange(_P, dtype=jnp.int32) // _K))
    ws = jnp.zeros((_PAD,), jnp.float32).at[pos].set(wflat)
    slot = pos
    block_expert = jnp.minimum(
        jnp.searchsorted(ends, jnp.arange(_NB) * _BM, side="right"),
        _E - 1).astype(jnp.int32)
    active = (jnp.arange(_NB) * _BM
              < (starts + counts)[block_expert]).astype(jnp.int32)
    return src, ws, block_expert, active, slot


def kernel(x, Wg, W1, b1, W2, b2):
    orig_shape = x.shape
    xt = x.reshape(_N, _D)
    idx8, w8 = _router(xt, Wg)
    src, ws, block_expert, active, slot = _dispatch_metadata(idx8, w8)
    xs = _sc_gather(xt, src)
    ys = _mlp(block_expert, active, xs, W1.astype(jnp.bfloat16),
              W2.astype(jnp.bfloat16), b1, b2, ws)
    out = _sc_combine(ys, slot)
    return out.reshape(orig_shape)


# speedup vs baseline: 1.8557x; 1.0928x over previous
import functools

import jax
import jax.numpy as jnp
from jax import lax
from jax.experimental import pallas as pl
from jax.experimental.pallas import tpu as pltpu
from jax.experimental.pallas import tpu_sc as plsc

_B, _S, _D = 2, 2048, 1024
_E, _K, _F = 8, 2, 4096
_N = _B * _S
_P = _N * _K

_BM = 256
_NB = _P // _BM + _E
_PAD = _NB * _BM

_BT = 512

_NC, _NS = 2, 16
_NW = _NC * _NS
_GCH = 64
_CT = 32


def _router_body(x_ref, wg_ref, idx_ref, w_ref):
    logits = jnp.dot(x_ref[...], wg_ref[...],
                     preferred_element_type=jnp.float32)
    col = lax.broadcasted_iota(jnp.int32, logits.shape, 1)
    m1 = jnp.max(logits, axis=1, keepdims=True)
    i1 = jnp.min(jnp.where(logits == m1, col, _E), axis=1, keepdims=True)
    l2 = jnp.where(col == i1, -jnp.inf, logits)
    m2 = jnp.max(l2, axis=1, keepdims=True)
    i2 = jnp.min(jnp.where(l2 == m2, col, _E), axis=1, keepdims=True)
    t = jnp.exp(m2 - m1)
    w1 = 1.0 / (1.0 + t)
    w2 = t / (1.0 + t)
    idx_ref[...] = jnp.where(col == 0, i1, jnp.where(col == 1, i2, 0))
    w_ref[...] = jnp.where(col == 0, w1, jnp.where(col == 1, w2, 0.0))


def _router(xt, wg):
    return pl.pallas_call(
        _router_body,
        grid=(_N // _BT,),
        in_specs=[pl.BlockSpec((_BT, _D), lambda i: (i, 0)),
                  pl.BlockSpec((_D, _E), lambda i: (0, 0))],
        out_specs=[pl.BlockSpec((_BT, _E), lambda i: (i, 0)),
                   pl.BlockSpec((_BT, _E), lambda i: (i, 0))],
        out_shape=[jax.ShapeDtypeStruct((_N, _E), jnp.int32),
                   jax.ShapeDtypeStruct((_N, _E), jnp.float32)],
    )(xt, wg)


def _mlp_body(be_ref, act_ref, src_ref, x_ref, w1_ref, w2_ref, b1_ref,
              b2_ref, ws_ref, y_ref):
    i = pl.program_id(0)

    @pl.when(act_ref[i] != 0)
    def _():
        e = be_ref[i]
        src = src_ref[pl.ds(i * _BM, _BM)]
        onehot = (src[:, None]
                  == lax.broadcasted_iota(jnp.int32, (_BM, _N), 1)
                  ).astype(jnp.bfloat16)
        x = jnp.dot(onehot, x_ref[...],
                    preferred_element_type=jnp.float32).astype(jnp.bfloat16)
        acc = jnp.broadcast_to(b2_ref[e, :][None, :], (_BM, _D))
        for jj in range(_F // 1024):
            sl = slice(jj * 1024, (jj + 1) * 1024)
            h = jnp.dot(x, w1_ref[0, :, sl],
                        preferred_element_type=jnp.float32)
            h += b1_ref[e, sl][None, :]
            h = jax.nn.gelu(h)
            acc = acc + jnp.dot(h.astype(jnp.bfloat16), w2_ref[0, sl, :],
                                preferred_element_type=jnp.float32)
        ws = ws_ref[pl.ds(i * _BM, _BM)]
        y_ref[...] = acc * ws[:, None]


def _mlp(block_expert, active, src, xb, w1, w2, b1, b2, ws):
    grid_spec = pltpu.PrefetchScalarGridSpec(
        num_scalar_prefetch=2,
        grid=(_NB,),
        in_specs=[
            pl.BlockSpec((_PAD,), lambda i, be, act: (0,)),
            pl.BlockSpec((_N, _D), lambda i, be, act: (0, 0)),
            pl.BlockSpec((1, _D, _F), lambda i, be, act: (be[i], 0, 0)),
            pl.BlockSpec((1, _F, _D), lambda i, be, act: (be[i], 0, 0)),
            pl.BlockSpec((_E, _F), lambda i, be, act: (0, 0)),
            pl.BlockSpec((_E, _D), lambda i, be, act: (0, 0)),
            pl.BlockSpec((_PAD,), lambda i, be, act: (0,)),
        ],
        out_specs=pl.BlockSpec((_BM, _D), lambda i, be, act: (i, 0)),
    )
    return pl.pallas_call(
        _mlp_body,
        grid_spec=grid_spec,
        out_shape=jax.ShapeDtypeStruct((_PAD, _D), jnp.float32),
        compiler_params=pltpu.CompilerParams(
            dimension_semantics=("arbitrary",)),
    )(block_expert, active, src, xb, w1, w2, b1, b2, ws)


def _sc_combine(ys, slots):
    tok_per_w = _N // _NW
    nch = tok_per_w // _CT
    mesh = plsc.VectorSubcoreMesh(core_axis_name="c", subcore_axis_name="s")

    @functools.partial(
        pl.kernel, mesh=mesh,
        out_type=jax.ShapeDtypeStruct((_N, _D), jnp.float32),
        scratch_types=[pltpu.VMEM((2 * _CT,), jnp.int32),
                       pltpu.VMEM((2 * _CT, _D), jnp.float32),
                       pltpu.VMEM((_CT, _D), jnp.float32),
                       pltpu.SemaphoreType.DMA])
    def ck(ys_hbm, sl_hbm, out_hbm, idx_v, rows_v, out_v, sem):
        wid = lax.axis_index("s") * _NC + lax.axis_index("c")
        t0 = wid * tok_per_w

        def chunk(c, carry):
            tb = t0 + c * _CT
            pltpu.sync_copy(sl_hbm.at[pl.ds(2 * tb, 2 * _CT)], idx_v)
            pltpu.async_copy(ys_hbm.at[idx_v], rows_v, sem).wait()

            def tok(r, carry2):
                def lane(q, carry3):
                    s = pl.ds(q * 16, 16)
                    out_v[r, s] = rows_v[2 * r, s] + rows_v[2 * r + 1, s]
                    return carry3
                return lax.fori_loop(0, _D // 16, lane, carry2)

            lax.fori_loop(0, _CT, tok, 0)
            pltpu.sync_copy(out_v, out_hbm.at[pl.ds(tb, _CT)])
            return carry

        lax.fori_loop(0, nch, chunk, 0)

    return ck(ys, slots)


def _dispatch_metadata(idx8, w8):
    eflat = idx8[:, :_K].reshape(-1)
    wflat = w8[:, :_K].reshape(-1)
    onehot = (eflat[:, None] == jnp.arange(_E)[None, :]).astype(jnp.int32)
    cum = jnp.cumsum(onehot, axis=0)
    counts = cum[-1]
    padded = ((counts + _BM - 1) // _BM) * _BM
    ends = jnp.cumsum(padded)
    starts = ends - padded
    rank = jnp.take_along_axis(cum, eflat[:, None], axis=1)[:, 0] - 1
    pos = (starts[eflat] + rank).astype(jnp.int32)
    src = jnp.zeros((_PAD,), jnp.int32).at[pos].set(
        (jnp.arange(_P, dtype=jnp.int32) // _K))
    ws = jnp.zeros((_PAD,), jnp.float32).at[pos].set(wflat)
    slot = pos
    block_expert = jnp.minimum(
        jnp.searchsorted(ends, jnp.arange(_NB) * _BM, side="right"),
        _E - 1).astype(jnp.int32)
    active = (jnp.arange(_NB) * _BM
              < (starts + counts)[block_expert]).astype(jnp.int32)
    return src, ws, block_expert, active, slot


def kernel(x, Wg, W1, b1, W2, b2):
    orig_shape = x.shape
    xt = x.reshape(_N, _D)
    idx8, w8 = _router(xt, Wg)
    src, ws, block_expert, active, slot = _dispatch_metadata(idx8, w8)
    ys = _mlp(block_expert, active, src, xt.astype(jnp.bfloat16),
              W1.astype(jnp.bfloat16), W2.astype(jnp.bfloat16), b1, b2, ws)
    out = _sc_combine(ys, slot)
    return out.reshape(orig_shape)


# speedup vs baseline: 1.9006x; 1.0242x over previous
import functools

import jax
import jax.numpy as jnp
from jax import lax
from jax.experimental import pallas as pl
from jax.experimental.pallas import tpu as pltpu
from jax.experimental.pallas import tpu_sc as plsc

_B, _S, _D = 2, 2048, 1024
_E, _K, _F = 8, 2, 4096
_N = _B * _S
_P = _N * _K

_BM = 256
_NB = _P // _BM + _E
_PAD = _NB * _BM

_BT = 512

_NC, _NS = 2, 16
_NW = _NC * _NS
_GCH = 64
_CT = 32


def _router_body(x_ref, wg_ref, idx_ref, w_ref, xb_ref):
    xv = x_ref[...]
    xb_ref[...] = xv.astype(jnp.bfloat16)
    logits = jnp.dot(xv, wg_ref[...],
                     preferred_element_type=jnp.float32)
    col = lax.broadcasted_iota(jnp.int32, logits.shape, 1)
    m1 = jnp.max(logits, axis=1, keepdims=True)
    i1 = jnp.min(jnp.where(logits == m1, col, _E), axis=1, keepdims=True)
    l2 = jnp.where(col == i1, -jnp.inf, logits)
    m2 = jnp.max(l2, axis=1, keepdims=True)
    i2 = jnp.min(jnp.where(l2 == m2, col, _E), axis=1, keepdims=True)
    t = jnp.exp(m2 - m1)
    w1 = 1.0 / (1.0 + t)
    w2 = t / (1.0 + t)
    idx_ref[...] = jnp.where(col == 0, i1, jnp.where(col == 1, i2, 0))
    w_ref[...] = jnp.where(col == 0, w1, jnp.where(col == 1, w2, 0.0))


def _router(xt, wg):
    return pl.pallas_call(
        _router_body,
        grid=(_N // _BT,),
        in_specs=[pl.BlockSpec((_BT, _D), lambda i: (i, 0)),
                  pl.BlockSpec((_D, _E), lambda i: (0, 0))],
        out_specs=[pl.BlockSpec((_BT, _E), lambda i: (i, 0)),
                   pl.BlockSpec((_BT, _E), lambda i: (i, 0)),
                   pl.BlockSpec((_BT, _D), lambda i: (i, 0))],
        out_shape=[jax.ShapeDtypeStruct((_N, _E), jnp.int32),
                   jax.ShapeDtypeStruct((_N, _E), jnp.float32),
                   jax.ShapeDtypeStruct((_N, _D), jnp.bfloat16)],
    )(xt, wg)


def _mlp_body(be_ref, act_ref, src_ref, x_ref, w1_ref, w2_ref, b1_ref,
              b2_ref, ws_ref, y_ref):
    i = pl.program_id(0)

    @pl.when(act_ref[i] != 0)
    def _():
        e = be_ref[i]
        src = src_ref[pl.ds(i * _BM, _BM)]
        onehot = (src[:, None]
                  == lax.broadcasted_iota(jnp.int32, (_BM, _N), 1)
                  ).astype(jnp.bfloat16)
        x = jnp.dot(onehot, x_ref[...],
                    preferred_element_type=jnp.float32).astype(jnp.bfloat16)
        acc = jnp.broadcast_to(b2_ref[e, :][None, :], (_BM, _D))
        for jj in range(_F // 4096):
            sl = slice(jj * 4096, (jj + 1) * 4096)
            h = jnp.dot(x, w1_ref[0, :, sl],
                        preferred_element_type=jnp.float32)
            h += b1_ref[e, sl][None, :]
            h = jax.nn.gelu(h)
            acc = acc + jnp.dot(h.astype(jnp.bfloat16), w2_ref[0, sl, :],
                                preferred_element_type=jnp.float32)
        ws = ws_ref[pl.ds(i * _BM, _BM)]
        y_ref[...] = acc * ws[:, None]


def _mlp(block_expert, active, src, xb, w1, w2, b1, b2, ws):
    grid_spec = pltpu.PrefetchScalarGridSpec(
        num_scalar_prefetch=2,
        grid=(_NB,),
        in_specs=[
            pl.BlockSpec((_PAD,), lambda i, be, act: (0,)),
            pl.BlockSpec((_N, _D), lambda i, be, act: (0, 0)),
            pl.BlockSpec((1, _D, _F), lambda i, be, act: (be[i], 0, 0)),
            pl.BlockSpec((1, _F, _D), lambda i, be, act: (be[i], 0, 0)),
            pl.BlockSpec((_E, _F), lambda i, be, act: (0, 0)),
            pl.BlockSpec((_E, _D), lambda i, be, act: (0, 0)),
            pl.BlockSpec((_PAD,), lambda i, be, act: (0,)),
        ],
        out_specs=pl.BlockSpec((_BM, _D), lambda i, be, act: (i, 0)),
    )
    return pl.pallas_call(
        _mlp_body,
        grid_spec=grid_spec,
        out_shape=jax.ShapeDtypeStruct((_PAD, _D), jnp.float32),
        compiler_params=pltpu.CompilerParams(
            dimension_semantics=("arbitrary",)),
    )(block_expert, active, src, xb, w1, w2, b1, b2, ws)


def _sc_combine(ys, slots):
    tok_per_w = _N // _NW
    nch = tok_per_w // _CT
    mesh = plsc.VectorSubcoreMesh(core_axis_name="c", subcore_axis_name="s")

    @functools.partial(
        pl.kernel, mesh=mesh,
        out_type=jax.ShapeDtypeStruct((_N, _D), jnp.float32),
        scratch_types=[pltpu.VMEM((2 * _CT,), jnp.int32),
                       pltpu.VMEM((2 * _CT, _D), jnp.float32),
                       pltpu.VMEM((_CT, _D), jnp.float32),
                       pltpu.SemaphoreType.DMA])
    def ck(ys_hbm, sl_hbm, out_hbm, idx_v, rows_v, out_v, sem):
        wid = lax.axis_index("s") * _NC + lax.axis_index("c")
        t0 = wid * tok_per_w

        def chunk(c, carry):
            tb = t0 + c * _CT
            pltpu.sync_copy(sl_hbm.at[pl.ds(2 * tb, 2 * _CT)], idx_v)
            pltpu.async_copy(ys_hbm.at[idx_v], rows_v, sem).wait()

            def tok(r, carry2):
                def lane(q, carry3):
                    s = pl.ds(q * 16, 16)
                    out_v[r, s] = rows_v[2 * r, s] + rows_v[2 * r + 1, s]
                    return carry3
                return lax.fori_loop(0, _D // 16, lane, carry2)

            lax.fori_loop(0, _CT, tok, 0)
            pltpu.sync_copy(out_v, out_hbm.at[pl.ds(tb, _CT)])
            return carry

        lax.fori_loop(0, nch, chunk, 0)

    return ck(ys, slots)


def _dispatch_metadata(idx8, w8):
    eflat = idx8[:, :_K].reshape(-1)
    wflat = w8[:, :_K].reshape(-1)
    onehot = (eflat[:, None] == jnp.arange(_E)[None, :]).astype(jnp.int32)
    cum = jnp.cumsum(onehot, axis=0)
    counts = cum[-1]
    padded = ((counts + _BM - 1) // _BM) * _BM
    ends = jnp.cumsum(padded)
    starts = ends - padded
    rank = jnp.take_along_axis(cum, eflat[:, None], axis=1)[:, 0] - 1
    pos = (starts[eflat] + rank).astype(jnp.int32)
    src = jnp.zeros((_PAD,), jnp.int32).at[pos].set(
        (jnp.arange(_P, dtype=jnp.int32) // _K))
    ws = jnp.zeros((_PAD,), jnp.float32).at[pos].set(wflat)
    slot = pos
    block_expert = jnp.minimum(
        jnp.searchsorted(ends, jnp.arange(_NB) * _BM, side="right"),
        _E - 1).astype(jnp.int32)
    active = (jnp.arange(_NB) * _BM
              < (starts + counts)[block_expert]).astype(jnp.int32)
    return src, ws, block_expert, active, slot


def kernel(x, Wg, W1, b1, W2, b2):
    orig_shape = x.shape
    xt = x.reshape(_N, _D)
    idx8, w8, xb = _router(xt, Wg)
    src, ws, block_expert, active, slot = _dispatch_metadata(idx8, w8)
    ys = _mlp(block_expert, active, src, xb,
              W1.astype(jnp.bfloat16), W2.astype(jnp.bfloat16), b1, b2, ws)
    out = _sc_combine(ys, slot)
    return out.reshape(orig_shape)


# speedup vs baseline: 1.9022x; 1.0008x over previous
import functools

import jax
import jax.numpy as jnp
from jax import lax
from jax.experimental import pallas as pl
from jax.experimental.pallas import tpu as pltpu
from jax.experimental.pallas import tpu_sc as plsc

_B, _S, _D = 2, 2048, 1024
_E, _K, _F = 8, 2, 4096
_N = _B * _S
_P = _N * _K

_BM = 256
_NB = _P // _BM + _E
_PAD = _NB * _BM

_BT = 512

_NC, _NS = 2, 16
_NW = _NC * _NS
_GCH = 64
_CT = 32


def _router_body(x_ref, wg_ref, idx_ref, w_ref, xb_ref):
    xv = x_ref[...]
    xb_ref[...] = xv.astype(jnp.bfloat16)
    logits = jnp.dot(xv, wg_ref[...],
                     preferred_element_type=jnp.float32)
    col = lax.broadcasted_iota(jnp.int32, logits.shape, 1)
    m1 = jnp.max(logits, axis=1, keepdims=True)
    i1 = jnp.min(jnp.where(logits == m1, col, _E), axis=1, keepdims=True)
    l2 = jnp.where(col == i1, -jnp.inf, logits)
    m2 = jnp.max(l2, axis=1, keepdims=True)
    i2 = jnp.min(jnp.where(l2 == m2, col, _E), axis=1, keepdims=True)
    t = jnp.exp(m2 - m1)
    w1 = 1.0 / (1.0 + t)
    w2 = t / (1.0 + t)
    idx_ref[...] = jnp.where(col == 0, i1, jnp.where(col == 1, i2, 0))
    w_ref[...] = jnp.where(col == 0, w1, jnp.where(col == 1, w2, 0.0))


def _router(xt, wg):
    return pl.pallas_call(
        _router_body,
        grid=(_N // _BT,),
        in_specs=[pl.BlockSpec((_BT, _D), lambda i: (i, 0)),
                  pl.BlockSpec((_D, _E), lambda i: (0, 0))],
        out_specs=[pl.BlockSpec((_BT, _E), lambda i: (i, 0)),
                   pl.BlockSpec((_BT, _E), lambda i: (i, 0)),
                   pl.BlockSpec((_BT, _D), lambda i: (i, 0))],
        out_shape=[jax.ShapeDtypeStruct((_N, _E), jnp.int32),
                   jax.ShapeDtypeStruct((_N, _E), jnp.float32),
                   jax.ShapeDtypeStruct((_N, _D), jnp.bfloat16)],
    )(xt, wg)


def _mlp_body(be_ref, act_ref, src_ref, x_ref, w1_ref, w2_ref, b1_ref,
              b2_ref, ws_ref, y_ref):
    i = pl.program_id(0)

    @pl.when(act_ref[i] != 0)
    def _():
        e = be_ref[i]
        src = src_ref[pl.ds(i * _BM, _BM)]
        onehot = (src[:, None]
                  == lax.broadcasted_iota(jnp.int32, (_BM, _N), 1)
                  ).astype(jnp.bfloat16)
        x = jnp.dot(onehot, x_ref[...],
                    preferred_element_type=jnp.float32).astype(jnp.bfloat16)
        acc = jnp.broadcast_to(b2_ref[e, :][None, :], (_BM, _D))
        for jj in range(_F // 4096):
            sl = slice(jj * 4096, (jj + 1) * 4096)
            h = jnp.dot(x, w1_ref[0, :, sl],
                        preferred_element_type=jnp.float32)
            h += b1_ref[e, sl][None, :]
            h = jax.nn.gelu(h)
            acc = acc + jnp.dot(h.astype(jnp.bfloat16), w2_ref[0, sl, :],
                                preferred_element_type=jnp.float32)
        ws = ws_ref[pl.ds(i * _BM, _BM)]
        y_ref[...] = acc * ws[:, None]


def _mlp(block_expert, active, src, xb, w1, w2, b1, b2, ws):
    grid_spec = pltpu.PrefetchScalarGridSpec(
        num_scalar_prefetch=2,
        grid=(_NB,),
        in_specs=[
            pl.BlockSpec((_PAD,), lambda i, be, act: (0,)),
            pl.BlockSpec((_N, _D), lambda i, be, act: (0, 0)),
            pl.BlockSpec((1, _D, _F), lambda i, be, act: (be[i], 0, 0)),
            pl.BlockSpec((1, _F, _D), lambda i, be, act: (be[i], 0, 0)),
            pl.BlockSpec((_E, _F), lambda i, be, act: (0, 0)),
            pl.BlockSpec((_E, _D), lambda i, be, act: (0, 0)),
            pl.BlockSpec((_PAD,), lambda i, be, act: (0,)),
        ],
        out_specs=pl.BlockSpec((_BM, _D), lambda i, be, act: (i, 0)),
    )
    return pl.pallas_call(
        _mlp_body,
        grid_spec=grid_spec,
        out_shape=jax.ShapeDtypeStruct((_PAD, _D), jnp.float32),
        compiler_params=pltpu.CompilerParams(
            dimension_semantics=("arbitrary",)),
    )(block_expert, active, src, xb, w1, w2, b1, b2, ws)


def _sc_combine(ys, slots):
    tok_per_w = _N // _NW
    nch = tok_per_w // _CT
    mesh = plsc.VectorSubcoreMesh(core_axis_name="c", subcore_axis_name="s")

    @functools.partial(
        pl.kernel, mesh=mesh,
        out_type=jax.ShapeDtypeStruct((_N, _D), jnp.float32),
        scratch_types=[pltpu.VMEM((2 * _CT,), jnp.int32),
                       pltpu.VMEM((2 * _CT, _D), jnp.float32),
                       pltpu.VMEM((_CT, _D), jnp.float32),
                       pltpu.SemaphoreType.DMA])
    def ck(ys_hbm, sl_hbm, out_hbm, idx_v, rows_v, out_v, sem):
        wid = lax.axis_index("s") * _NC + lax.axis_index("c")
        t0 = wid * tok_per_w

        def chunk(c, carry):
            tb = t0 + c * _CT
            pltpu.sync_copy(sl_hbm.at[pl.ds(2 * tb, 2 * _CT)], idx_v)
            pltpu.async_copy(ys_hbm.at[idx_v], rows_v, sem).wait()

            def tok(r, carry2):
                def lane(q, carry3):
                    for u in range(4):
                        s = pl.ds((q * 4 + u) * 16, 16)
                        out_v[r, s] = rows_v[2 * r, s] + rows_v[2 * r + 1, s]
                    return carry3
                return lax.fori_loop(0, _D // 64, lane, carry2)

            lax.fori_loop(0, _CT, tok, 0)
            pltpu.sync_copy(out_v, out_hbm.at[pl.ds(tb, _CT)])
            return carry

        lax.fori_loop(0, nch, chunk, 0)

    return ck(ys, slots)


def _dispatch_metadata(idx8, w8):
    eflat = idx8[:, :_K].reshape(-1)
    wflat = w8[:, :_K].reshape(-1)
    onehot = (eflat[:, None] == jnp.arange(_E)[None, :]).astype(jnp.int32)
    cum = jnp.cumsum(onehot, axis=0)
    counts = cum[-1]
    padded = ((counts + _BM - 1) // _BM) * _BM
    ends = jnp.cumsum(padded)
    starts = ends - padded
    rank = jnp.take_along_axis(cum, eflat[:, None], axis=1)[:, 0] - 1
    pos = (starts[eflat] + rank).astype(jnp.int32)
    src = jnp.zeros((_PAD,), jnp.int32).at[pos].set(
        (jnp.arange(_P, dtype=jnp.int32) // _K))
    ws = jnp.zeros((_PAD,), jnp.float32).at[pos].set(wflat)
    slot = pos
    block_expert = jnp.minimum(
        jnp.searchsorted(ends, jnp.arange(_NB) * _BM, side="right"),
        _E - 1).astype(jnp.int32)
    active = (jnp.arange(_NB) * _BM
              < (starts + counts)[block_expert]).astype(jnp.int32)
    return src, ws, block_expert, active, slot


def kernel(x, Wg, W1, b1, W2, b2):
    orig_shape = x.shape
    xt = x.reshape(_N, _D)
    idx8, w8, xb = _router(xt, Wg)
    src, ws, block_expert, active, slot = _dispatch_metadata(idx8, w8)
    ys = _mlp(block_expert, active, src, xb,
              W1.astype(jnp.bfloat16), W2.astype(jnp.bfloat16), b1, b2, ws)
    out = _sc_combine(ys, slot)
    return out.reshape(orig_shape)
